# baseline (speedup 1.0000x reference)
import jax
import jax.numpy as jnp
from jax import lax
from jax.experimental import pallas as pl
from jax.experimental.pallas import tpu as pltpu
from jax.experimental.pallas import tpu_sc as plsc

NUM_CORES = 2
NUM_SUBCORES = 16
NUM_WORKERS = NUM_CORES * NUM_SUBCORES

CHUNK = 512


def _gather_kernel(n_rows, d):
    rows_per_w = n_rows // NUM_WORKERS
    n_chunks = rows_per_w // CHUNK
    mesh = plsc.VectorSubcoreMesh(core_axis_name="c", subcore_axis_name="s")

    def body(idx_hbm, table_hbm, out_hbm, idx_v, rows_v, sem):
        wid = lax.axis_index("s") * NUM_CORES + lax.axis_index("c")
        wbase = wid * rows_per_w

        def step(i, carry):
            base = wbase + i * CHUNK
            pltpu.sync_copy(idx_hbm.at[pl.ds(base, CHUNK)], idx_v)
            pltpu.async_copy(table_hbm.at[idx_v], rows_v, sem).wait()
            pltpu.sync_copy(rows_v, out_hbm.at[pl.ds(base, CHUNK)])
            return carry

        lax.fori_loop(0, n_chunks, step, 0)

    return pl.kernel(
        body,
        out_type=jax.ShapeDtypeStruct((n_rows, d), jnp.float32),
        mesh=mesh,
        scratch_types=[
            pltpu.VMEM((CHUNK,), jnp.int32),
            pltpu.VMEM((CHUNK, d), jnp.float32),
            pltpu.SemaphoreType.DMA,
        ],
        compiler_params=pltpu.CompilerParams(use_tc_tiling_on_sc=False),
    )


def kernel(x, emb_table):
    b, s = x.shape
    _, d = emb_table.shape
    idx = x.reshape(b * s)
    out = _gather_kernel(b * s, d)(idx, emb_table)
    return out.reshape(b, s, d)


# speedup vs baseline: 1.0434x; 1.0434x over previous
import jax
import jax.numpy as jnp
from jax import lax
from jax.experimental import pallas as pl
from jax.experimental.pallas import tpu as pltpu
from jax.experimental.pallas import tpu_sc as plsc

NUM_CORES = 2
NUM_SUBCORES = 16
NUM_WORKERS = NUM_CORES * NUM_SUBCORES

CHUNK = 320
NBUF = 4


def _gather_kernel(n_rows, d):
    rows_per_w = n_rows // NUM_WORKERS
    n_chunks = rows_per_w // CHUNK
    n_groups = n_chunks // NBUF
    mesh = plsc.VectorSubcoreMesh(core_axis_name="c", subcore_axis_name="s")

    def body(idx_hbm, table_hbm, out_hbm, idx_v, rows_v, *sems):
        sem_g = sems[:NBUF]
        sem_w = sems[NBUF:]
        wid = lax.axis_index("s") * NUM_CORES + lax.axis_index("c")
        wbase = wid * rows_per_w

        def start_gather(c, slot):
            pltpu.async_copy(table_hbm.at[idx_v.at[c]], rows_v.at[slot],
                             sem_g[slot])

        def wait_gather(slot):
            pltpu.make_async_copy(table_hbm.at[idx_v.at[0]], rows_v.at[slot],
                                  sem_g[slot]).wait()

        def start_write(c, slot):
            pltpu.async_copy(rows_v.at[slot],
                             out_hbm.at[pl.ds(wbase + c * CHUNK, CHUNK)],
                             sem_w[slot])

        def wait_write(slot):
            pltpu.make_async_copy(rows_v.at[slot],
                                  out_hbm.at[pl.ds(wbase, CHUNK)],
                                  sem_w[slot]).wait()

        pltpu.sync_copy(idx_hbm.at[pl.ds(wid * n_chunks, n_chunks)], idx_v)

        start_gather(0, 0)
        start_gather(1, 1)

        def group(q, carry):
            for b in range(NBUF):
                c = q * NBUF + b
                wait_gather(b)
                start_write(c, b)
                nxt = (b + 2) % NBUF
                if b < 2:
                    @pl.when(q > 0)
                    def _():
                        wait_write(nxt)
                    start_gather(c + 2, nxt)
                else:
                    wait_write(nxt)

                    @pl.when(q < n_groups - 1)
                    def _():
                        start_gather(c + 2, nxt)
            return carry

        lax.fori_loop(0, n_groups, group, 0)

        wait_write(2)
        wait_write(3)

    return pl.kernel(
        body,
        out_type=jax.ShapeDtypeStruct((n_rows, d), jnp.float32),
        mesh=mesh,
        scratch_types=[
            pltpu.VMEM((n_chunks, CHUNK), jnp.int32),
            pltpu.VMEM((NBUF, CHUNK, d), jnp.float32),
        ] + [pltpu.SemaphoreType.DMA] * (2 * NBUF),
        compiler_params=pltpu.CompilerParams(use_tc_tiling_on_sc=False),
    )


def kernel(x, emb_table):
    b, s = x.shape
    _, d = emb_table.shape
    n = b * s
    idx = x.reshape(n // CHUNK, CHUNK)
    out = _gather_kernel(n, d)(idx, emb_table)
    return out.reshape(b, s, d)


# speedup vs baseline: 1.0456x; 1.0021x over previous
import jax
import jax.numpy as jnp
from jax import lax
from jax.experimental import pallas as pl
from jax.experimental.pallas import tpu as pltpu
from jax.experimental.pallas import tpu_sc as plsc

NUM_CORES = 2
NUM_SUBCORES = 16
NUM_WORKERS = NUM_CORES * NUM_SUBCORES

NBUF = 4


def _gather_kernel(b, s, d):
    rows_per_w = b // NUM_WORKERS
    n_groups = rows_per_w // NBUF
    mesh = plsc.VectorSubcoreMesh(core_axis_name="c", subcore_axis_name="s")

    def body(idx_hbm, table_hbm, out_hbm, idx_v, rows_v, *sems):
        sem_g = sems[:NBUF]
        sem_w = sems[NBUF:]
        wid = lax.axis_index("s") * NUM_CORES + lax.axis_index("c")
        wbase = wid * rows_per_w

        def start_gather(c, slot):
            pltpu.async_copy(table_hbm.at[idx_v.at[c]], rows_v.at[slot],
                             sem_g[slot])

        def wait_gather(slot):
            pltpu.make_async_copy(table_hbm.at[idx_v.at[0]], rows_v.at[slot],
                                  sem_g[slot]).wait()

        def start_write(c, slot):
            pltpu.async_copy(rows_v.at[slot], out_hbm.at[wbase + c],
                             sem_w[slot])

        def wait_write(slot):
            pltpu.make_async_copy(rows_v.at[slot], out_hbm.at[wbase],
                                  sem_w[slot]).wait()

        pltpu.sync_copy(idx_hbm.at[pl.ds(wbase, rows_per_w)], idx_v)

        start_gather(0, 0)
        start_gather(1, 1)

        def group(q, carry):
            for bslot in range(NBUF):
                c = q * NBUF + bslot
                wait_gather(bslot)
                start_write(c, bslot)
                nxt = (bslot + 2) % NBUF
                if bslot < 2:
                    @pl.when(q > 0)
                    def _():
                        wait_write(nxt)
                    start_gather(c + 2, nxt)
                else:
                    wait_write(nxt)

                    @pl.when(q < n_groups - 1)
                    def _():
                        start_gather(c + 2, nxt)
            return carry

        lax.fori_loop(0, n_groups, group, 0)

        wait_write(2)
        wait_write(3)

    return pl.kernel(
        body,
        out_type=jax.ShapeDtypeStruct((b, s, d), jnp.float32),
        mesh=mesh,
        scratch_types=[
            pltpu.VMEM((rows_per_w, s), jnp.int32),
            pltpu.VMEM((NBUF, s, d), jnp.float32),
        ] + [pltpu.SemaphoreType.DMA] * (2 * NBUF),
        compiler_params=pltpu.CompilerParams(use_tc_tiling_on_sc=False),
    )


def kernel(x, emb_table):
    b, s = x.shape
    _, d = emb_table.shape
    return _gather_kernel(b, s, d)(x, emb_table)


# speedup vs baseline: 1.2750x; 1.2194x over previous
import jax
import jax.numpy as jnp
from jax import lax
from jax.experimental import pallas as pl
from jax.experimental.pallas import tpu as pltpu
from jax.experimental.pallas import tpu_sc as plsc

NUM_CORES = 2
NUM_SUBCORES = 16
NUM_WORKERS = NUM_CORES * NUM_SUBCORES

CHUNK = 128
NBUF = 4


def _gather_kernel(n, d2):
    rows_per_w = n // NUM_WORKERS
    n_chunks = rows_per_w // CHUNK
    n_groups = n_chunks // NBUF
    mesh = plsc.VectorSubcoreMesh(core_axis_name="c", subcore_axis_name="s")

    def body(idx_hbm, table_hbm, out_hbm, idx_v, rows_v, *sems):
        sem_g = sems[:NBUF]
        sem_w = sems[NBUF:]
        wid = lax.axis_index("s") * NUM_CORES + lax.axis_index("c")
        wbase = wid * rows_per_w

        def start_gather(c, slot):
            pltpu.async_copy(table_hbm.at[idx_v.at[pl.ds(c * CHUNK, CHUNK)]],
                             rows_v.at[slot], sem_g[slot])

        def wait_gather(slot):
            pltpu.make_async_copy(table_hbm.at[idx_v.at[pl.ds(0, CHUNK)]],
                                  rows_v.at[slot], sem_g[slot]).wait()

        def start_write(c, slot):
            pltpu.async_copy(rows_v.at[slot],
                             out_hbm.at[pl.ds(wbase + c * CHUNK, CHUNK)],
                             sem_w[slot])

        def wait_write(slot):
            pltpu.make_async_copy(rows_v.at[slot],
                                  out_hbm.at[pl.ds(wbase, CHUNK)],
                                  sem_w[slot]).wait()

        pltpu.sync_copy(idx_hbm.at[pl.ds(wbase, rows_per_w)], idx_v)

        start_gather(0, 0)
        start_gather(1, 1)

        def group(q, carry):
            for bslot in range(NBUF):
                c = q * NBUF + bslot
                wait_gather(bslot)
                start_write(c, bslot)
                nxt = (bslot + 2) % NBUF
                if bslot < 2:
                    @pl.when(q > 0)
                    def _():
                        wait_write(nxt)
                    start_gather(c + 2, nxt)
                else:
                    wait_write(nxt)

                    @pl.when(q < n_groups - 1)
                    def _():
                        start_gather(c + 2, nxt)
            return carry

        lax.fori_loop(0, n_groups, group, 0)

        wait_write(2)
        wait_write(3)

    return pl.kernel(
        body,
        out_type=jax.ShapeDtypeStruct((n, d2), jnp.float32),
        mesh=mesh,
        scratch_types=[
            pltpu.VMEM((rows_per_w,), jnp.int32),
            pltpu.VMEM((NBUF, CHUNK, d2), jnp.float32),
        ] + [pltpu.SemaphoreType.DMA] * (2 * NBUF),
        compiler_params=pltpu.CompilerParams(use_tc_tiling_on_sc=True),
    )


def kernel(x, emb_table):
    b, s = x.shape
    v, d = emb_table.shape
    n = b * s
    t128 = jnp.pad(emb_table, ((0, 0), (0, 128 - d)))
    out128 = _gather_kernel(n, 128)(x.reshape(n), t128)
    return out128[:, :d].reshape(b, s, d)
